# baseline (speedup 1.0000x reference)
import functools

import jax
import jax.numpy as jnp
from jax import lax
from jax.experimental import pallas as pl
from jax.experimental.pallas import tpu as pltpu
from jax.experimental.pallas import tpu_sc as plsc

N = 10000
E = 320000
EU = 160000
D = 128
H = 128
TF = 16

NC = 2
NS = 16
NW = NC * NS
LANES = 16
LG = H // LANES

EPW = E // NW
EUPW = EU // NW
MC = 40
MCH = EPW // MC
UCH = EUPW // MC
NBUF = 3
NIB = 2 * NBUF

FLUSH_TILES = 10
FLUSH_ROWS = 1000


@functools.cache
def _mesh():
    return plsc.VectorSubcoreMesh(
        core_axis_name="c", subcore_axis_name="s",
        num_cores=NC, num_subcores=NS,
    )


def _sc_body(with_c, zw_hbm, c_hbm, ei_hbm, out_hbm, refs):
    rbufs = refs[0:NBUF]
    if with_c:
        cbufs = refs[NBUF:2 * NBUF]
        rest = refs[2 * NBUF:]
    else:
        cbufs = None
        rest = refs[NBUF:]
    ibufs = rest[0:NIB]
    acc_sh = rest[NIB]
    gsems = rest[NIB + 1:NIB + 1 + NBUF]
    if with_c:
        csems = rest[NIB + 1 + NBUF:NIB + 1 + 2 * NBUF]
        isems = rest[NIB + 1 + 2 * NBUF:]
    else:
        csems = None
        isems = rest[NIB + 1 + NBUF:]

    cid = lax.axis_index("c")
    sid = lax.axis_index("s")
    wid = sid * NC + cid

    @pl.when(sid < FLUSH_TILES)
    def _():
        zero = jnp.zeros((LANES,), jnp.float32)

        def zrow(g, _):
            rbufs[0][g // LG, pl.ds((g % LG) * LANES, LANES)] = zero
            return 0

        lax.fori_loop(0, MC * LG, zrow, 0)
        for t in range(FLUSH_ROWS // MC):
            pltpu.sync_copy(
                rbufs[0], acc_sh.at[pl.ds(sid * FLUSH_ROWS + t * MC, MC)]
            )

    plsc.subcore_barrier()

    def run_pass(nch, ibase, compute):
        def fetch(g, r):
            pltpu.async_copy(
                ei_hbm.at[pl.ds(ibase + g, 1)], ibufs[r], isems[r]
            )

        def gather(g, r, b):
            pltpu.make_async_copy(
                ei_hbm.at[pl.ds(ibase, 1)], ibufs[r], isems[r]
            ).wait()
            pltpu.async_copy(
                zw_hbm.at[ibufs[r].at[0, 0]], rbufs[b], gsems[b]
            )
            if with_c:
                pltpu.async_copy(
                    c_hbm.at[pl.ds(wid * EPW + g * MC, MC)],
                    cbufs[b], csems[b],
                )

        def wait_rows(b):
            pltpu.make_async_copy(
                zw_hbm.at[ibufs[0].at[0, 0]], rbufs[b], gsems[b]
            ).wait()
            if with_c:
                pltpu.make_async_copy(
                    c_hbm.at[pl.ds(0, MC)], cbufs[b], csems[b]
                ).wait()

        def scatter(r, b):
            pltpu.sync_copy(
                rbufs[b], acc_sh.at[ibufs[r].at[0, 1]], add=True
            )

        for g in range(min(NIB, nch)):
            fetch(g, g % NIB)
        for g in range(min(NBUF, nch)):
            gather(g, g % NIB, g % NBUF)

        def step(g, r, b, r_nxt, guard):
            wait_rows(b)
            if compute is not None:
                compute(b)
            scatter(r, b)
            nxt = g + NBUF
            nxt2 = g + NIB

            def advance():
                gather(nxt, r_nxt, b)

            def refetch():
                fetch(nxt2, r)

            if guard:
                if nxt < nch:
                    advance()
                if nxt2 < nch:
                    refetch()
            else:
                @pl.when(nxt < nch)
                def _():
                    advance()

                @pl.when(nxt2 < nch)
                def _():
                    refetch()

        def body(s, _):
            for k in range(NIB):
                g = s * NIB + k
                step(g, k, k % NBUF, (k + NBUF) % NIB, guard=False)
            return 0

        lax.fori_loop(0, nch // NIB, body, 0)
        for g in range(nch - nch % NIB, nch):
            step(g, g % NIB, g % NBUF, (g + NBUF) % NIB, guard=True)

    if with_c:
        def compute_a(b):
            def row(r, _):
                for j in range(LG):
                    col = j * LANES
                    v = rbufs[b][r, pl.ds(col, LANES)] \
                        + cbufs[b][r, pl.ds(col, LANES)]
                    rbufs[b][r, pl.ds(col, LANES)] = jnp.maximum(v, 0.0)
                return 0

            lax.fori_loop(0, MC, row, 0)

        run_pass(MCH, wid * MCH, compute_a)
    else:
        run_pass(UCH, wid * UCH, None)

    plsc.subcore_barrier()

    @pl.when(sid < FLUSH_TILES)
    def _():
        base = sid * FLUSH_ROWS
        pltpu.sync_copy(
            acc_sh.at[pl.ds(base, FLUSH_ROWS)],
            out_hbm.at[cid, pl.ds(base, FLUSH_ROWS)],
        )


def _sc_msg_body(zw_hbm, c_hbm, ei_hbm, out_hbm, *refs):
    _sc_body(True, zw_hbm, c_hbm, ei_hbm, out_hbm, refs)


def _sc_uniq_body(zu_hbm, ue_hbm, out_hbm, *refs):
    _sc_body(False, zu_hbm, None, ue_hbm, out_hbm, refs)


@functools.cache
def _sc_msg():
    sems = [pltpu.SemaphoreType.DMA] * (NBUF + NBUF + NIB)
    return pl.kernel(
        _sc_msg_body,
        out_type=jax.ShapeDtypeStruct((NC, N, H), jnp.float32),
        mesh=_mesh(),
        scratch_types=(
            [pltpu.VMEM((MC, H), jnp.float32)] * NBUF
            + [pltpu.VMEM((MC, H), jnp.float32)] * NBUF
            + [pltpu.VMEM((1, 2, MC), jnp.int32)] * NIB
            + [pltpu.VMEM_SHARED((N, H), jnp.float32)]
            + sems
        ),
    )


@functools.cache
def _sc_uniq():
    sems = [pltpu.SemaphoreType.DMA] * (NBUF + NIB)
    return pl.kernel(
        _sc_uniq_body,
        out_type=jax.ShapeDtypeStruct((NC, N, H), jnp.float32),
        mesh=_mesh(),
        scratch_types=(
            [pltpu.VMEM((MC, H), jnp.float32)] * NBUF
            + [pltpu.VMEM((1, 2, MC), jnp.int32)] * NIB
            + [pltpu.VMEM_SHARED((N, H), jnp.float32)]
            + sems
        ),
    )


_EB = 2000
_NB = 2000


_PI_HI = 3.140625
_PI_LO = 9.67653589793e-4
_INV_PI = 0.3183098861837907
_S1 = -1.6666667163e-01
_S2 = 8.3333337680e-03
_S3 = -1.9841270114e-04
_S4 = 2.7557314297e-06


def _fast_sin(u):
    k = jnp.floor(u * _INV_PI + 0.5)
    x = u - k * _PI_HI - k * _PI_LO
    x2 = x * x
    p = x * (1.0 + x2 * (_S1 + x2 * (_S2 + x2 * (_S3 + x2 * _S4))))
    odd = (k.astype(jnp.int32) & 1) == 1
    return jnp.where(odd, -p, p)


def _edge_const_body(tsl_ref, td_ref, s0_ref, tew_ref, teb_ref,
                     wp_ref, wl_ref, wd_ref, bm_ref, c_ref):
    t = tsl_ref[0, 0, :]
    lin = (s0_ref[0, 0] * t + s0_ref[0, 1])[:, None]
    per = _fast_sin(
        t[:, None] * tew_ref[0, :][None, :] + teb_ref[0, :][None, :]
    )
    c = jnp.dot(per, wp_ref[...], preferred_element_type=jnp.float32)
    c += lin * wl_ref[0, :][None, :]
    c += td_ref[0, 0, :][:, None] * wd_ref[0, :][None, :]
    c_ref[...] = c + bm_ref[0, :][None, :]


def _edge_const(tsl, td, te_w0, te_b0, te_w, te_b, Wt, bm):
    grid = E // _EB
    s0 = jnp.stack([te_w0, te_b0]).reshape(1, 2)
    return pl.pallas_call(
        _edge_const_body,
        grid=(grid,),
        in_specs=[
            pl.BlockSpec((1, 1, _EB), lambda i: (i, 0, 0)),
            pl.BlockSpec((1, 1, _EB), lambda i: (i, 0, 0)),
            pl.BlockSpec((1, 2), lambda i: (0, 0)),
            pl.BlockSpec((1, TF - 1), lambda i: (0, 0)),
            pl.BlockSpec((1, TF - 1), lambda i: (0, 0)),
            pl.BlockSpec((TF - 1, H), lambda i: (0, 0)),
            pl.BlockSpec((1, H), lambda i: (0, 0)),
            pl.BlockSpec((1, H), lambda i: (0, 0)),
            pl.BlockSpec((1, H), lambda i: (0, 0)),
        ],
        out_specs=pl.BlockSpec((_EB, H), lambda i: (i, 0)),
        out_shape=jax.ShapeDtypeStruct((E, H), jnp.float32),
    )(tsl.reshape(grid, 1, _EB), td.reshape(grid, 1, _EB), s0,
      te_w.reshape(1, TF - 1), te_b.reshape(1, TF - 1),
      Wt[1:TF], Wt[0].reshape(1, H), Wt[TF].reshape(1, H),
      bm.reshape(1, H))


def _pre_nodes_body(x_ref, wm_ref, wu_ref, zw_ref, zu_ref):
    xb = x_ref[...]
    zw_ref[...] = jnp.dot(xb, wm_ref[...], preferred_element_type=jnp.float32)
    zu_ref[...] = jnp.dot(xb, wu_ref[...], preferred_element_type=jnp.float32)


def _pre_nodes(xm, Wmx, Wu):
    return pl.pallas_call(
        _pre_nodes_body,
        grid=(N // _NB,),
        in_specs=[
            pl.BlockSpec((_NB, D), lambda i: (i, 0)),
            pl.BlockSpec((D, H), lambda i: (0, 0)),
            pl.BlockSpec((D, H), lambda i: (0, 0)),
        ],
        out_specs=[
            pl.BlockSpec((_NB, H), lambda i: (i, 0)),
            pl.BlockSpec((_NB, H), lambda i: (i, 0)),
        ],
        out_shape=[
            jax.ShapeDtypeStruct((N, H), jnp.float32),
            jax.ShapeDtypeStruct((N, H), jnp.float32),
        ],
    )(xm, Wmx, Wu)


def _mid_body(acca_ref, accu_ref, z_ref, ws_ref, bo_ref, wm_ref, wu_ref,
              z1_ref, zw_ref, zu_ref):
    a = acca_ref[0] + acca_ref[1] + accu_ref[0] + accu_ref[1]
    a += jnp.dot(z_ref[...], ws_ref[...], preferred_element_type=jnp.float32)
    z1 = jnp.maximum(a + bo_ref[0, :][None, :], 0.0)
    z1_ref[...] = z1
    zw_ref[...] = jnp.dot(z1, wm_ref[...], preferred_element_type=jnp.float32)
    zu_ref[...] = jnp.dot(z1, wu_ref[...], preferred_element_type=jnp.float32)


def _mid(acca, accu, z, Ws, bo, Wmx, Wu):
    return pl.pallas_call(
        _mid_body,
        grid=(N // _NB,),
        in_specs=[
            pl.BlockSpec((NC, _NB, H), lambda i: (0, i, 0)),
            pl.BlockSpec((NC, _NB, H), lambda i: (0, i, 0)),
            pl.BlockSpec((_NB, D), lambda i: (i, 0)),
            pl.BlockSpec((D, H), lambda i: (0, 0)),
            pl.BlockSpec((1, H), lambda i: (0, 0)),
            pl.BlockSpec((D, H), lambda i: (0, 0)),
            pl.BlockSpec((D, H), lambda i: (0, 0)),
        ],
        out_specs=[
            pl.BlockSpec((_NB, H), lambda i: (i, 0)),
            pl.BlockSpec((_NB, H), lambda i: (i, 0)),
            pl.BlockSpec((_NB, H), lambda i: (i, 0)),
        ],
        out_shape=[
            jax.ShapeDtypeStruct((N, H), jnp.float32),
            jax.ShapeDtypeStruct((N, H), jnp.float32),
            jax.ShapeDtypeStruct((N, H), jnp.float32),
        ],
    )(acca, accu, z, Ws, bo.reshape(1, H), Wmx, Wu)


def _final_body(acca_ref, accu_ref, z_ref, ws_ref, bo_ref, o_ref):
    a = acca_ref[0] + acca_ref[1] + accu_ref[0] + accu_ref[1]
    a += jnp.dot(z_ref[...], ws_ref[...], preferred_element_type=jnp.float32)
    o_ref[...] = jnp.maximum(a + bo_ref[0, :][None, :], 0.0)


def _final(acca, accu, z, Ws, bo):
    return pl.pallas_call(
        _final_body,
        grid=(N // _NB,),
        in_specs=[
            pl.BlockSpec((NC, _NB, H), lambda i: (0, i, 0)),
            pl.BlockSpec((NC, _NB, H), lambda i: (0, i, 0)),
            pl.BlockSpec((_NB, D), lambda i: (i, 0)),
            pl.BlockSpec((D, H), lambda i: (0, 0)),
            pl.BlockSpec((1, H), lambda i: (0, 0)),
        ],
        out_specs=pl.BlockSpec((_NB, H), lambda i: (i, 0)),
        out_shape=jax.ShapeDtypeStruct((N, H), jnp.float32),
    )(acca, accu, z, Ws, bo.reshape(1, H))


def kernel(x, edge_index, timestamps, time_diffs, unique_edges,
           timestamp_lists, te_w0, te_b0, te_w, te_b,
           Wm0, bm0, Wu0, Ws0, bo0, Wm1, bm1, Wu1, Ws1, bo1):
    sdr = jnp.swapaxes(edge_index.reshape(2, NW * MCH, MC), 0, 1)
    usdr = jnp.swapaxes(unique_edges.reshape(2, NW * UCH, MC), 0, 1)

    zw0, zu0 = _pre_nodes(x, Wm0[:D], Wu0)
    accu0 = _sc_uniq()(zu0, usdr)
    c0 = _edge_const(timestamp_lists, time_diffs, te_w0, te_b0,
                     te_w, te_b, Wm0[D:], bm0)
    sdr0 = sdr + accu0[0, 0, 0].astype(jnp.int32) * 0
    acca0 = _sc_msg()(zw0, c0, sdr0)
    c1 = _edge_const(timestamp_lists, time_diffs, te_w0, te_b0,
                     te_w, te_b, Wm1[D:], bm1)
    z1, zw1, zu1 = _mid(acca0, accu0, x, Ws0, bo0, Wm1[:D], Wu1)
    accu1 = _sc_uniq()(zu1, usdr)
    sdr1 = sdr + accu1[0, 0, 0].astype(jnp.int32) * 0
    acca1 = _sc_msg()(zw1, c1, sdr1)
    return _final(acca1, accu1, z1, Ws1, bo1)


# speedup vs baseline: 1.0293x; 1.0293x over previous
import functools

import jax
import jax.numpy as jnp
from jax import lax
from jax.experimental import pallas as pl
from jax.experimental.pallas import tpu as pltpu
from jax.experimental.pallas import tpu_sc as plsc

N = 10000
E = 320000
EU = 160000
D = 128
H = 128
TF = 16

NC = 2
NS = 16
NW = NC * NS
LANES = 16
LG = H // LANES

EPW = E // NW
EUPW = EU // NW
MC = 40
MCH = EPW // MC
UCH = EUPW // MC
NBUF = 3
NIB = 2 * NBUF

FLUSH_TILES = 10
FLUSH_ROWS = 1000


@functools.cache
def _mesh():
    return plsc.VectorSubcoreMesh(
        core_axis_name="c", subcore_axis_name="s",
        num_cores=NC, num_subcores=NS,
    )


def _sc_layer_body(zw_hbm, zu_hbm, c_hbm, ei_hbm, ue_hbm, out_hbm, *refs):
    rbufs = refs[0:NBUF]
    cbufs = refs[NBUF:2 * NBUF]
    ibufs = refs[2 * NBUF:2 * NBUF + NIB]
    ubufs = refs[2 * NBUF + NIB:2 * NBUF + 2 * NIB]
    acc_sh = refs[2 * NBUF + 2 * NIB]
    s0 = 2 * NBUF + 2 * NIB + 1
    gsems = refs[s0:s0 + NBUF]
    csems = refs[s0 + NBUF:s0 + 2 * NBUF]
    isems = refs[s0 + 2 * NBUF:s0 + 2 * NBUF + NIB]
    usems = refs[s0 + 2 * NBUF + NIB:]

    cid = lax.axis_index("c")
    sid = lax.axis_index("s")
    wid = sid * NC + cid

    @pl.when(sid < FLUSH_TILES)
    def _():
        zero = jnp.zeros((LANES,), jnp.float32)

        def zrow(g, _):
            rbufs[0][g // LG, pl.ds((g % LG) * LANES, LANES)] = zero
            return 0

        lax.fori_loop(0, MC * LG, zrow, 0)
        for t in range(FLUSH_ROWS // MC):
            pltpu.sync_copy(
                rbufs[0], acc_sh.at[pl.ds(sid * FLUSH_ROWS + t * MC, MC)]
            )

    plsc.subcore_barrier()

    def run_pass(nch, ibase, idx_hbm, idx_bufs, idx_sems, tab_hbm, with_c,
                 compute):
        def fetch(g, r):
            pltpu.async_copy(
                idx_hbm.at[pl.ds(ibase + g, 1)], idx_bufs[r], idx_sems[r]
            )

        def gather(g, r, b):
            pltpu.make_async_copy(
                idx_hbm.at[pl.ds(ibase, 1)], idx_bufs[r], idx_sems[r]
            ).wait()
            pltpu.async_copy(
                tab_hbm.at[idx_bufs[r].at[0, 0]], rbufs[b], gsems[b]
            )
            if with_c:
                pltpu.async_copy(
                    c_hbm.at[pl.ds(wid * EPW + g * MC, MC)],
                    cbufs[b], csems[b],
                )

        def wait_rows(b):
            pltpu.make_async_copy(
                tab_hbm.at[idx_bufs[0].at[0, 0]], rbufs[b], gsems[b]
            ).wait()
            if with_c:
                pltpu.make_async_copy(
                    c_hbm.at[pl.ds(0, MC)], cbufs[b], csems[b]
                ).wait()

        def scatter(r, b):
            pltpu.sync_copy(
                rbufs[b], acc_sh.at[idx_bufs[r].at[0, 1]], add=True
            )

        for g in range(min(NIB, nch)):
            fetch(g, g % NIB)
        for g in range(min(NBUF, nch)):
            gather(g, g % NIB, g % NBUF)

        def step(g, r, b, r_nxt, guard):
            wait_rows(b)
            if compute is not None:
                compute(b)
            scatter(r, b)
            nxt = g + NBUF
            nxt2 = g + NIB

            def advance():
                gather(nxt, r_nxt, b)

            def refetch():
                fetch(nxt2, r)

            if guard:
                if nxt < nch:
                    advance()
                if nxt2 < nch:
                    refetch()
            else:
                @pl.when(nxt < nch)
                def _():
                    advance()

                @pl.when(nxt2 < nch)
                def _():
                    refetch()

        def body(s, _):
            for k in range(NIB):
                g = s * NIB + k
                step(g, k, k % NBUF, (k + NBUF) % NIB, guard=False)
            return 0

        lax.fori_loop(0, nch // NIB, body, 0)
        for g in range(nch - nch % NIB, nch):
            step(g, g % NIB, g % NBUF, (g + NBUF) % NIB, guard=True)

    def compute_a(b):
        def row(r, _):
            for j in range(LG):
                col = j * LANES
                v = rbufs[b][r, pl.ds(col, LANES)] \
                    + cbufs[b][r, pl.ds(col, LANES)]
                rbufs[b][r, pl.ds(col, LANES)] = jnp.maximum(v, 0.0)
            return 0

        lax.fori_loop(0, MC, row, 0)

    run_pass(MCH, wid * MCH, ei_hbm, ibufs, isems, zw_hbm, True, compute_a)

    run_pass(UCH, wid * UCH, ue_hbm, ubufs, usems, zu_hbm, False, None)

    plsc.subcore_barrier()

    @pl.when(sid < FLUSH_TILES)
    def _():
        base = sid * FLUSH_ROWS
        pltpu.sync_copy(
            acc_sh.at[pl.ds(base, FLUSH_ROWS)],
            out_hbm.at[cid, pl.ds(base, FLUSH_ROWS)],
        )


@functools.cache
def _sc_layer():
    sems = [pltpu.SemaphoreType.DMA] * (NBUF + NBUF + NIB + NIB)
    return pl.kernel(
        _sc_layer_body,
        out_type=jax.ShapeDtypeStruct((NC, N, H), jnp.float32),
        mesh=_mesh(),
        scratch_types=(
            [pltpu.VMEM((MC, H), jnp.float32)] * NBUF
            + [pltpu.VMEM((MC, H), jnp.float32)] * NBUF
            + [pltpu.VMEM((1, 2, MC), jnp.int32)] * NIB
            + [pltpu.VMEM((1, 2, MC), jnp.int32)] * NIB
            + [pltpu.VMEM_SHARED((N, H), jnp.float32)]
            + sems
        ),
    )


_EB = 2000
_NB = 2000


_PI_HI = 3.140625
_PI_LO = 9.67653589793e-4
_INV_PI = 0.3183098861837907
_S1 = -1.6666667163e-01
_S2 = 8.3333337680e-03
_S3 = -1.9841270114e-04
_S4 = 2.7557314297e-06


def _fast_sin(u):
    k = jnp.floor(u * _INV_PI + 0.5)
    x = u - k * _PI_HI - k * _PI_LO
    x2 = x * x
    p = x * (1.0 + x2 * (_S1 + x2 * (_S2 + x2 * (_S3 + x2 * _S4))))
    odd = (k.astype(jnp.int32) & 1) == 1
    return jnp.where(odd, -p, p)


def _edge_const_body(tsl_ref, td_ref, s0_ref, tew_ref, teb_ref,
                     wp_ref, wl_ref, wd_ref, bm_ref, c_ref):
    t = tsl_ref[0, 0, :]
    lin = (s0_ref[0, 0] * t + s0_ref[0, 1])[:, None]
    per = _fast_sin(
        t[:, None] * tew_ref[0, :][None, :] + teb_ref[0, :][None, :]
    )
    c = jnp.dot(per, wp_ref[...], preferred_element_type=jnp.float32)
    c += lin * wl_ref[0, :][None, :]
    c += td_ref[0, 0, :][:, None] * wd_ref[0, :][None, :]
    c_ref[...] = c + bm_ref[0, :][None, :]


def _edge_const(tsl, td, te_w0, te_b0, te_w, te_b, Wt, bm):
    grid = E // _EB
    s0 = jnp.stack([te_w0, te_b0]).reshape(1, 2)
    return pl.pallas_call(
        _edge_const_body,
        grid=(grid,),
        in_specs=[
            pl.BlockSpec((1, 1, _EB), lambda i: (i, 0, 0)),
            pl.BlockSpec((1, 1, _EB), lambda i: (i, 0, 0)),
            pl.BlockSpec((1, 2), lambda i: (0, 0)),
            pl.BlockSpec((1, TF - 1), lambda i: (0, 0)),
            pl.BlockSpec((1, TF - 1), lambda i: (0, 0)),
            pl.BlockSpec((TF - 1, H), lambda i: (0, 0)),
            pl.BlockSpec((1, H), lambda i: (0, 0)),
            pl.BlockSpec((1, H), lambda i: (0, 0)),
            pl.BlockSpec((1, H), lambda i: (0, 0)),
        ],
        out_specs=pl.BlockSpec((_EB, H), lambda i: (i, 0)),
        out_shape=jax.ShapeDtypeStruct((E, H), jnp.float32),
    )(tsl.reshape(grid, 1, _EB), td.reshape(grid, 1, _EB), s0,
      te_w.reshape(1, TF - 1), te_b.reshape(1, TF - 1),
      Wt[1:TF], Wt[0].reshape(1, H), Wt[TF].reshape(1, H),
      bm.reshape(1, H))


def _pre_nodes_body(x_ref, wm_ref, wu_ref, zw_ref, zu_ref):
    xb = x_ref[...]
    zw_ref[...] = jnp.dot(xb, wm_ref[...], preferred_element_type=jnp.float32)
    zu_ref[...] = jnp.dot(xb, wu_ref[...], preferred_element_type=jnp.float32)


def _pre_nodes(xm, Wmx, Wu):
    return pl.pallas_call(
        _pre_nodes_body,
        grid=(N // _NB,),
        in_specs=[
            pl.BlockSpec((_NB, D), lambda i: (i, 0)),
            pl.BlockSpec((D, H), lambda i: (0, 0)),
            pl.BlockSpec((D, H), lambda i: (0, 0)),
        ],
        out_specs=[
            pl.BlockSpec((_NB, H), lambda i: (i, 0)),
            pl.BlockSpec((_NB, H), lambda i: (i, 0)),
        ],
        out_shape=[
            jax.ShapeDtypeStruct((N, H), jnp.float32),
            jax.ShapeDtypeStruct((N, H), jnp.float32),
        ],
    )(xm, Wmx, Wu)


def _mid_body(acc_ref, z_ref, ws_ref, bo_ref, wm_ref, wu_ref,
              z1_ref, zw_ref, zu_ref):
    a = acc_ref[0] + acc_ref[1]
    a += jnp.dot(z_ref[...], ws_ref[...], preferred_element_type=jnp.float32)
    z1 = jnp.maximum(a + bo_ref[0, :][None, :], 0.0)
    z1_ref[...] = z1
    zw_ref[...] = jnp.dot(z1, wm_ref[...], preferred_element_type=jnp.float32)
    zu_ref[...] = jnp.dot(z1, wu_ref[...], preferred_element_type=jnp.float32)


def _mid(acc, z, Ws, bo, Wmx, Wu):
    return pl.pallas_call(
        _mid_body,
        grid=(N // _NB,),
        in_specs=[
            pl.BlockSpec((NC, _NB, H), lambda i: (0, i, 0)),
            pl.BlockSpec((_NB, D), lambda i: (i, 0)),
            pl.BlockSpec((D, H), lambda i: (0, 0)),
            pl.BlockSpec((1, H), lambda i: (0, 0)),
            pl.BlockSpec((D, H), lambda i: (0, 0)),
            pl.BlockSpec((D, H), lambda i: (0, 0)),
        ],
        out_specs=[
            pl.BlockSpec((_NB, H), lambda i: (i, 0)),
            pl.BlockSpec((_NB, H), lambda i: (i, 0)),
            pl.BlockSpec((_NB, H), lambda i: (i, 0)),
        ],
        out_shape=[
            jax.ShapeDtypeStruct((N, H), jnp.float32),
            jax.ShapeDtypeStruct((N, H), jnp.float32),
            jax.ShapeDtypeStruct((N, H), jnp.float32),
        ],
    )(acc, z, Ws, bo.reshape(1, H), Wmx, Wu)


def _final_body(acc_ref, z_ref, ws_ref, bo_ref, o_ref):
    a = acc_ref[0] + acc_ref[1]
    a += jnp.dot(z_ref[...], ws_ref[...], preferred_element_type=jnp.float32)
    o_ref[...] = jnp.maximum(a + bo_ref[0, :][None, :], 0.0)


def _final(acc, z, Ws, bo):
    return pl.pallas_call(
        _final_body,
        grid=(N // _NB,),
        in_specs=[
            pl.BlockSpec((NC, _NB, H), lambda i: (0, i, 0)),
            pl.BlockSpec((_NB, D), lambda i: (i, 0)),
            pl.BlockSpec((D, H), lambda i: (0, 0)),
            pl.BlockSpec((1, H), lambda i: (0, 0)),
        ],
        out_specs=pl.BlockSpec((_NB, H), lambda i: (i, 0)),
        out_shape=jax.ShapeDtypeStruct((N, H), jnp.float32),
    )(acc, z, Ws, bo.reshape(1, H))


def kernel(x, edge_index, timestamps, time_diffs, unique_edges,
           timestamp_lists, te_w0, te_b0, te_w, te_b,
           Wm0, bm0, Wu0, Ws0, bo0, Wm1, bm1, Wu1, Ws1, bo1):
    sdr = jnp.swapaxes(edge_index.reshape(2, NW * MCH, MC), 0, 1)
    usdr = jnp.swapaxes(unique_edges.reshape(2, NW * UCH, MC), 0, 1)

    c0 = _edge_const(timestamp_lists, time_diffs, te_w0, te_b0,
                     te_w, te_b, Wm0[D:], bm0)
    zw0, zu0 = _pre_nodes(x, Wm0[:D], Wu0)
    acc0 = _sc_layer()(zw0, zu0, c0, sdr, usdr)
    c1 = _edge_const(timestamp_lists, time_diffs, te_w0, te_b0,
                     te_w, te_b, Wm1[D:], bm1)
    z1, zw1, zu1 = _mid(acc0, x, Ws0, bo0, Wm1[:D], Wu1)
    acc1 = _sc_layer()(zw1, zu1, c1, sdr, usdr)
    return _final(acc1, z1, Ws1, bo1)


# speedup vs baseline: 1.0342x; 1.0048x over previous
import functools

import jax
import jax.numpy as jnp
from jax import lax
from jax.experimental import pallas as pl
from jax.experimental.pallas import tpu as pltpu
from jax.experimental.pallas import tpu_sc as plsc

N = 10000
E = 320000
EU = 160000
D = 128
H = 128
TF = 16

NC = 2
NS = 16
NW = NC * NS
LANES = 16
LG = H // LANES

EPW = E // NW
EUPW = EU // NW
MC = 40
MCH = EPW // MC
UCH = EUPW // MC
NBUF = 3
NIB = 2 * NBUF

FLUSH_TILES = 10
FLUSH_ROWS = 1000


@functools.cache
def _mesh():
    return plsc.VectorSubcoreMesh(
        core_axis_name="c", subcore_axis_name="s",
        num_cores=NC, num_subcores=NS,
    )


def _sc_layer_body(zw_hbm, zu_hbm, c_hbm, ei_hbm, ue_hbm, out_hbm, *refs):
    rbufs = refs[0:NBUF]
    cbufs = refs[NBUF:2 * NBUF]
    ibufs = refs[2 * NBUF:2 * NBUF + NIB]
    ubufs = refs[2 * NBUF + NIB:2 * NBUF + 2 * NIB]
    acc_sh = refs[2 * NBUF + 2 * NIB]
    s0 = 2 * NBUF + 2 * NIB + 1
    gsems = refs[s0:s0 + NBUF]
    csems = refs[s0 + NBUF:s0 + 2 * NBUF]
    isems = refs[s0 + 2 * NBUF:s0 + 2 * NBUF + NIB]
    usems = refs[s0 + 2 * NBUF + NIB:]

    cid = lax.axis_index("c")
    sid = lax.axis_index("s")
    wid = sid * NC + cid

    @pl.when(sid < FLUSH_TILES)
    def _():
        zero = jnp.zeros((LANES,), jnp.float32)

        def zrow(g, _):
            rbufs[0][g // LG, pl.ds((g % LG) * LANES, LANES)] = zero
            return 0

        lax.fori_loop(0, MC * LG, zrow, 0)
        for t in range(FLUSH_ROWS // MC):
            pltpu.sync_copy(
                rbufs[0], acc_sh.at[pl.ds(sid * FLUSH_ROWS + t * MC, MC)]
            )

    plsc.subcore_barrier()

    def run_pass(nch, ibase, idx_hbm, idx_bufs, idx_sems, tab_hbm, with_c,
                 compute):
        def fetch(g, r):
            pltpu.async_copy(
                idx_hbm.at[pl.ds(ibase + g, 1)], idx_bufs[r], idx_sems[r]
            )

        def gather(g, r, b):
            pltpu.make_async_copy(
                idx_hbm.at[pl.ds(ibase, 1)], idx_bufs[r], idx_sems[r]
            ).wait()
            pltpu.async_copy(
                tab_hbm.at[idx_bufs[r].at[0, 0]], rbufs[b], gsems[b]
            )
            if with_c:
                pltpu.async_copy(
                    c_hbm.at[pl.ds(wid * EPW + g * MC, MC)],
                    cbufs[b], csems[b],
                )

        def wait_rows(b):
            pltpu.make_async_copy(
                tab_hbm.at[idx_bufs[0].at[0, 0]], rbufs[b], gsems[b]
            ).wait()
            if with_c:
                pltpu.make_async_copy(
                    c_hbm.at[pl.ds(0, MC)], cbufs[b], csems[b]
                ).wait()

        def scatter(r, b):
            pltpu.sync_copy(
                rbufs[b], acc_sh.at[idx_bufs[r].at[0, 1]], add=True
            )

        for g in range(min(NIB, nch)):
            fetch(g, g % NIB)
        for g in range(min(NBUF, nch)):
            gather(g, g % NIB, g % NBUF)

        def step(g, r, b, r_nxt, guard):
            wait_rows(b)
            if compute is not None:
                compute(b)
            scatter(r, b)
            nxt = g + NBUF
            nxt2 = g + NIB

            def advance():
                gather(nxt, r_nxt, b)

            def refetch():
                fetch(nxt2, r)

            if guard:
                if nxt < nch:
                    advance()
                if nxt2 < nch:
                    refetch()
            else:
                @pl.when(nxt < nch)
                def _():
                    advance()

                @pl.when(nxt2 < nch)
                def _():
                    refetch()

        def body(s, _):
            for k in range(NIB):
                g = s * NIB + k
                step(g, k, k % NBUF, (k + NBUF) % NIB, guard=False)
            return 0

        lax.fori_loop(0, nch // NIB, body, 0)
        for g in range(nch - nch % NIB, nch):
            step(g, g % NIB, g % NBUF, (g + NBUF) % NIB, guard=True)

    def compute_a(b):
        def row(r, _):
            for j in range(LG):
                col = j * LANES
                v = rbufs[b][r, pl.ds(col, LANES)] \
                    + cbufs[b][r, pl.ds(col, LANES)]
                rbufs[b][r, pl.ds(col, LANES)] = jnp.maximum(v, 0.0)
            return 0

        lax.fori_loop(0, MC, row, 0)

    run_pass(MCH, wid * MCH, ei_hbm, ibufs, isems, zw_hbm, True, compute_a)

    run_pass(UCH, wid * UCH, ue_hbm, ubufs, usems, zu_hbm, False, None)

    plsc.subcore_barrier()

    @pl.when(sid < FLUSH_TILES)
    def _():
        base = sid * FLUSH_ROWS
        pltpu.sync_copy(
            acc_sh.at[pl.ds(base, FLUSH_ROWS)],
            out_hbm.at[cid, pl.ds(base, FLUSH_ROWS)],
        )


@functools.cache
def _sc_layer():
    sems = [pltpu.SemaphoreType.DMA] * (NBUF + NBUF + NIB + NIB)
    return pl.kernel(
        _sc_layer_body,
        out_type=jax.ShapeDtypeStruct((NC, N, H), jnp.float32),
        mesh=_mesh(),
        scratch_types=(
            [pltpu.VMEM((MC, H), jnp.float32)] * NBUF
            + [pltpu.VMEM((MC, H), jnp.float32)] * NBUF
            + [pltpu.VMEM((1, 2, MC), jnp.int32)] * NIB
            + [pltpu.VMEM((1, 2, MC), jnp.int32)] * NIB
            + [pltpu.VMEM_SHARED((N, H), jnp.float32)]
            + sems
        ),
    )


_EB = 2000
_NB = 2000


_PI_HI = 3.140625
_PI_LO = 9.67653589793e-4
_INV_PI = 0.3183098861837907
_S1 = -1.6666667163e-01
_S2 = 8.3333337680e-03
_S3 = -1.9841270114e-04
_S4 = 2.7557314297e-06


def _fast_sin(u):
    k = jnp.floor(u * _INV_PI + 0.5)
    x = u - k * _PI_HI - k * _PI_LO
    x2 = x * x
    p = x * (1.0 + x2 * (_S1 + x2 * (_S2 + x2 * (_S3 + x2 * _S4))))
    odd = (k.astype(jnp.int32) & 1) == 1
    return jnp.where(odd, -p, p)


def _edge_const_body(tsl_ref, td_ref, s0_ref, tew_ref, teb_ref,
                     wt_ref, bm_ref, c_ref):
    t = tsl_ref[0, 0, :]
    lin = (s0_ref[0, 0] * t + s0_ref[0, 1])[:, None]
    per = _fast_sin(
        t[:, None] * tew_ref[0, :][None, :] + teb_ref[0, :][None, :]
    )
    feats = jnp.concatenate([lin, per, td_ref[0, 0, :][:, None]], axis=1)
    c_ref[...] = (
        jnp.dot(feats, wt_ref[...], preferred_element_type=jnp.float32)
        + bm_ref[0, :][None, :]
    )


def _edge_const(tsl, td, te_w0, te_b0, te_w, te_b, Wt, bm):
    grid = E // _EB
    s0 = jnp.stack([te_w0, te_b0]).reshape(1, 2)
    return pl.pallas_call(
        _edge_const_body,
        grid=(grid,),
        in_specs=[
            pl.BlockSpec((1, 1, _EB), lambda i: (i, 0, 0)),
            pl.BlockSpec((1, 1, _EB), lambda i: (i, 0, 0)),
            pl.BlockSpec((1, 2), lambda i: (0, 0)),
            pl.BlockSpec((1, TF - 1), lambda i: (0, 0)),
            pl.BlockSpec((1, TF - 1), lambda i: (0, 0)),
            pl.BlockSpec((TF + 1, H), lambda i: (0, 0)),
            pl.BlockSpec((1, H), lambda i: (0, 0)),
        ],
        out_specs=pl.BlockSpec((_EB, H), lambda i: (i, 0)),
        out_shape=jax.ShapeDtypeStruct((E, H), jnp.float32),
    )(tsl.reshape(grid, 1, _EB), td.reshape(grid, 1, _EB), s0,
      te_w.reshape(1, TF - 1), te_b.reshape(1, TF - 1), Wt,
      bm.reshape(1, H))


def _pre_nodes_body(x_ref, wm_ref, wu_ref, zw_ref, zu_ref):
    xb = x_ref[...]
    zw_ref[...] = jnp.dot(xb, wm_ref[...], preferred_element_type=jnp.float32)
    zu_ref[...] = jnp.dot(xb, wu_ref[...], preferred_element_type=jnp.float32)


def _pre_nodes(xm, Wmx, Wu):
    return pl.pallas_call(
        _pre_nodes_body,
        grid=(N // _NB,),
        in_specs=[
            pl.BlockSpec((_NB, D), lambda i: (i, 0)),
            pl.BlockSpec((D, H), lambda i: (0, 0)),
            pl.BlockSpec((D, H), lambda i: (0, 0)),
        ],
        out_specs=[
            pl.BlockSpec((_NB, H), lambda i: (i, 0)),
            pl.BlockSpec((_NB, H), lambda i: (i, 0)),
        ],
        out_shape=[
            jax.ShapeDtypeStruct((N, H), jnp.float32),
            jax.ShapeDtypeStruct((N, H), jnp.float32),
        ],
    )(xm, Wmx, Wu)


def _mid_body(acc_ref, z_ref, ws_ref, bo_ref, wm_ref, wu_ref,
              z1_ref, zw_ref, zu_ref):
    a = acc_ref[0] + acc_ref[1]
    a += jnp.dot(z_ref[...], ws_ref[...], preferred_element_type=jnp.float32)
    z1 = jnp.maximum(a + bo_ref[0, :][None, :], 0.0)
    z1_ref[...] = z1
    zw_ref[...] = jnp.dot(z1, wm_ref[...], preferred_element_type=jnp.float32)
    zu_ref[...] = jnp.dot(z1, wu_ref[...], preferred_element_type=jnp.float32)


def _mid(acc, z, Ws, bo, Wmx, Wu):
    return pl.pallas_call(
        _mid_body,
        grid=(N // _NB,),
        in_specs=[
            pl.BlockSpec((NC, _NB, H), lambda i: (0, i, 0)),
            pl.BlockSpec((_NB, D), lambda i: (i, 0)),
            pl.BlockSpec((D, H), lambda i: (0, 0)),
            pl.BlockSpec((1, H), lambda i: (0, 0)),
            pl.BlockSpec((D, H), lambda i: (0, 0)),
            pl.BlockSpec((D, H), lambda i: (0, 0)),
        ],
        out_specs=[
            pl.BlockSpec((_NB, H), lambda i: (i, 0)),
            pl.BlockSpec((_NB, H), lambda i: (i, 0)),
            pl.BlockSpec((_NB, H), lambda i: (i, 0)),
        ],
        out_shape=[
            jax.ShapeDtypeStruct((N, H), jnp.float32),
            jax.ShapeDtypeStruct((N, H), jnp.float32),
            jax.ShapeDtypeStruct((N, H), jnp.float32),
        ],
    )(acc, z, Ws, bo.reshape(1, H), Wmx, Wu)


def _final_body(acc_ref, z_ref, ws_ref, bo_ref, o_ref):
    a = acc_ref[0] + acc_ref[1]
    a += jnp.dot(z_ref[...], ws_ref[...], preferred_element_type=jnp.float32)
    o_ref[...] = jnp.maximum(a + bo_ref[0, :][None, :], 0.0)


def _final(acc, z, Ws, bo):
    return pl.pallas_call(
        _final_body,
        grid=(N // _NB,),
        in_specs=[
            pl.BlockSpec((NC, _NB, H), lambda i: (0, i, 0)),
            pl.BlockSpec((_NB, D), lambda i: (i, 0)),
            pl.BlockSpec((D, H), lambda i: (0, 0)),
            pl.BlockSpec((1, H), lambda i: (0, 0)),
        ],
        out_specs=pl.BlockSpec((_NB, H), lambda i: (i, 0)),
        out_shape=jax.ShapeDtypeStruct((N, H), jnp.float32),
    )(acc, z, Ws, bo.reshape(1, H))


def kernel(x, edge_index, timestamps, time_diffs, unique_edges,
           timestamp_lists, te_w0, te_b0, te_w, te_b,
           Wm0, bm0, Wu0, Ws0, bo0, Wm1, bm1, Wu1, Ws1, bo1):
    sdr = jnp.swapaxes(edge_index.reshape(2, NW * MCH, MC), 0, 1)
    usdr = jnp.swapaxes(unique_edges.reshape(2, NW * UCH, MC), 0, 1)

    c0 = _edge_const(timestamp_lists, time_diffs, te_w0, te_b0,
                     te_w, te_b, Wm0[D:], bm0)
    zw0, zu0 = _pre_nodes(x, Wm0[:D], Wu0)
    acc0 = _sc_layer()(zw0, zu0, c0, sdr, usdr)
    c1 = _edge_const(timestamp_lists, time_diffs, te_w0, te_b0,
                     te_w, te_b, Wm1[D:], bm1)
    z1, zw1, zu1 = _mid(acc0, x, Ws0, bo0, Wm1[:D], Wu1)
    acc1 = _sc_layer()(zw1, zu1, c1, sdr, usdr)
    return _final(acc1, z1, Ws1, bo1)


# speedup vs baseline: 1.0805x; 1.0448x over previous
import functools

import jax
import jax.numpy as jnp
from jax import lax
from jax.experimental import pallas as pl
from jax.experimental.pallas import tpu as pltpu
from jax.experimental.pallas import tpu_sc as plsc

N = 10000
E = 320000
EU = 160000
D = 128
H = 128
TF = 16

NC = 2
NS = 16
NW = NC * NS
LANES = 16
LG = H // LANES

EPW = E // NW
EUPW = EU // NW
MC = 40
MCH = EPW // MC
UCH = EUPW // MC
NBUF = 3
NIB = 2 * NBUF

FLUSH_TILES = 10
FLUSH_ROWS = 1000


@functools.cache
def _mesh():
    return plsc.VectorSubcoreMesh(
        core_axis_name="c", subcore_axis_name="s",
        num_cores=NC, num_subcores=NS,
    )


def _sc_layer_body(zw_hbm, zu_hbm, c_hbm, src_hbm, dst_hbm,
                   usrc_hbm, udst_hbm, out_hbm, *refs):
    rbufs = refs[0:NBUF]
    cbufs = refs[NBUF:2 * NBUF]
    sbufs = refs[2 * NBUF:2 * NBUF + NIB]
    dbufs = refs[2 * NBUF + NIB:2 * NBUF + 2 * NIB]
    usbufs = refs[2 * NBUF + 2 * NIB:2 * NBUF + 3 * NIB]
    udbufs = refs[2 * NBUF + 3 * NIB:2 * NBUF + 4 * NIB]
    acc_sh = refs[2 * NBUF + 4 * NIB]
    s0 = 2 * NBUF + 4 * NIB + 1
    gsems = refs[s0:s0 + NBUF]
    csems = refs[s0 + NBUF:s0 + 2 * NBUF]
    isems = refs[s0 + 2 * NBUF:s0 + 2 * NBUF + NIB]
    usems = refs[s0 + 2 * NBUF + NIB:]

    cid = lax.axis_index("c")
    sid = lax.axis_index("s")
    wid = sid * NC + cid

    @pl.when(sid < FLUSH_TILES)
    def _():
        zero = jnp.zeros((LANES,), jnp.float32)

        def zrow(g, _):
            rbufs[0][g // LG, pl.ds((g % LG) * LANES, LANES)] = zero
            return 0

        lax.fori_loop(0, MC * LG, zrow, 0)
        for t in range(FLUSH_ROWS // MC):
            pltpu.sync_copy(
                rbufs[0], acc_sh.at[pl.ds(sid * FLUSH_ROWS + t * MC, MC)]
            )

    plsc.subcore_barrier()

    def run_pass(nch, ibase, src_hbm, dst_hbm, sb, db, idx_sems, tab_hbm,
                 with_c, compute):
        def fetch(g, r):
            base = ibase + g * MC
            pltpu.async_copy(
                src_hbm.at[pl.ds(base, MC)], sb[r], idx_sems[r]
            )
            pltpu.async_copy(
                dst_hbm.at[pl.ds(base, MC)], db[r], idx_sems[r]
            )

        def gather(g, r, b):
            pltpu.make_async_copy(
                src_hbm.at[pl.ds(ibase, MC)], sb[r], idx_sems[r]
            ).wait()
            pltpu.make_async_copy(
                dst_hbm.at[pl.ds(ibase, MC)], db[r], idx_sems[r]
            ).wait()
            pltpu.async_copy(tab_hbm.at[sb[r]], rbufs[b], gsems[b])
            if with_c:
                pltpu.async_copy(
                    c_hbm.at[pl.ds(wid * EPW + g * MC, MC)],
                    cbufs[b], csems[b],
                )

        def wait_rows(b):
            pltpu.make_async_copy(
                tab_hbm.at[sb[0]], rbufs[b], gsems[b]
            ).wait()
            if with_c:
                pltpu.make_async_copy(
                    c_hbm.at[pl.ds(0, MC)], cbufs[b], csems[b]
                ).wait()

        def scatter(r, b):
            pltpu.sync_copy(rbufs[b], acc_sh.at[db[r]], add=True)

        for g in range(min(NIB, nch)):
            fetch(g, g % NIB)
        for g in range(min(NBUF, nch)):
            gather(g, g % NIB, g % NBUF)

        def step(g, r, b, r_nxt, guard):
            wait_rows(b)
            if compute is not None:
                compute(b)
            scatter(r, b)
            nxt = g + NBUF
            nxt2 = g + NIB

            def advance():
                gather(nxt, r_nxt, b)

            def refetch():
                fetch(nxt2, r)

            if guard:
                if nxt < nch:
                    advance()
                if nxt2 < nch:
                    refetch()
            else:
                @pl.when(nxt < nch)
                def _():
                    advance()

                @pl.when(nxt2 < nch)
                def _():
                    refetch()

        def body(s, _):
            for k in range(NIB):
                g = s * NIB + k
                step(g, k, k % NBUF, (k + NBUF) % NIB, guard=False)
            return 0

        lax.fori_loop(0, nch // NIB, body, 0)
        for g in range(nch - nch % NIB, nch):
            step(g, g % NIB, g % NBUF, (g + NBUF) % NIB, guard=True)

    def compute_a(b):
        def row(r, _):
            for j in range(LG):
                col = j * LANES
                v = rbufs[b][r, pl.ds(col, LANES)] \
                    + cbufs[b][r, pl.ds(col, LANES)]
                rbufs[b][r, pl.ds(col, LANES)] = jnp.maximum(v, 0.0)
            return 0

        lax.fori_loop(0, MC, row, 0)

    run_pass(MCH, wid * EPW, src_hbm, dst_hbm, sbufs, dbufs, isems,
             zw_hbm, True, compute_a)

    run_pass(UCH, wid * EUPW, usrc_hbm, udst_hbm, usbufs, udbufs, usems,
             zu_hbm, False, None)

    plsc.subcore_barrier()

    @pl.when(sid < FLUSH_TILES)
    def _():
        base = sid * FLUSH_ROWS
        pltpu.sync_copy(
            acc_sh.at[pl.ds(base, FLUSH_ROWS)],
            out_hbm.at[cid, pl.ds(base, FLUSH_ROWS)],
        )


@functools.cache
def _sc_layer():
    sems = [pltpu.SemaphoreType.DMA] * (NBUF + NBUF + NIB + NIB)
    return pl.kernel(
        _sc_layer_body,
        out_type=jax.ShapeDtypeStruct((NC, N, H), jnp.float32),
        mesh=_mesh(),
        scratch_types=(
            [pltpu.VMEM((MC, H), jnp.float32)] * NBUF
            + [pltpu.VMEM((MC, H), jnp.float32)] * NBUF
            + [pltpu.VMEM((MC,), jnp.int32)] * (4 * NIB)
            + [pltpu.VMEM_SHARED((N, H), jnp.float32)]
            + sems
        ),
    )


_EB = 2000
_NB = 2000


_PI_HI = 3.140625
_PI_LO = 9.67653589793e-4
_INV_PI = 0.3183098861837907
_S1 = -1.6666667163e-01
_S2 = 8.3333337680e-03
_S3 = -1.9841270114e-04
_S4 = 2.7557314297e-06


def _fast_sin(u):
    k = jnp.floor(u * _INV_PI + 0.5)
    x = u - k * _PI_HI - k * _PI_LO
    x2 = x * x
    p = x * (1.0 + x2 * (_S1 + x2 * (_S2 + x2 * (_S3 + x2 * _S4))))
    odd = (k.astype(jnp.int32) & 1) == 1
    return jnp.where(odd, -p, p)


def _edge_const_body(tsl_ref, td_ref, s0_ref, tew_ref, teb_ref,
                     wt_ref, bm_ref, c_ref):
    t = tsl_ref[0, 0, :]
    lin = (s0_ref[0, 0] * t + s0_ref[0, 1])[:, None]
    per = _fast_sin(
        t[:, None] * tew_ref[0, :][None, :] + teb_ref[0, :][None, :]
    )
    feats = jnp.concatenate([lin, per, td_ref[0, 0, :][:, None]], axis=1)
    c_ref[...] = (
        jnp.dot(feats, wt_ref[...], preferred_element_type=jnp.float32)
        + bm_ref[0, :][None, :]
    )


def _edge_const(tsl, td, te_w0, te_b0, te_w, te_b, Wt, bm):
    grid = E // _EB
    s0 = jnp.stack([te_w0, te_b0]).reshape(1, 2)
    return pl.pallas_call(
        _edge_const_body,
        grid=(grid,),
        in_specs=[
            pl.BlockSpec((1, 1, _EB), lambda i: (i, 0, 0)),
            pl.BlockSpec((1, 1, _EB), lambda i: (i, 0, 0)),
            pl.BlockSpec((1, 2), lambda i: (0, 0)),
            pl.BlockSpec((1, TF - 1), lambda i: (0, 0)),
            pl.BlockSpec((1, TF - 1), lambda i: (0, 0)),
            pl.BlockSpec((TF + 1, H), lambda i: (0, 0)),
            pl.BlockSpec((1, H), lambda i: (0, 0)),
        ],
        out_specs=pl.BlockSpec((_EB, H), lambda i: (i, 0)),
        out_shape=jax.ShapeDtypeStruct((E, H), jnp.float32),
    )(tsl.reshape(grid, 1, _EB), td.reshape(grid, 1, _EB), s0,
      te_w.reshape(1, TF - 1), te_b.reshape(1, TF - 1), Wt,
      bm.reshape(1, H))


def _pre_nodes_body(x_ref, wm_ref, wu_ref, zw_ref, zu_ref):
    xb = x_ref[...]
    zw_ref[...] = jnp.dot(xb, wm_ref[...], preferred_element_type=jnp.float32)
    zu_ref[...] = jnp.dot(xb, wu_ref[...], preferred_element_type=jnp.float32)


def _pre_nodes(xm, Wmx, Wu):
    return pl.pallas_call(
        _pre_nodes_body,
        grid=(N // _NB,),
        in_specs=[
            pl.BlockSpec((_NB, D), lambda i: (i, 0)),
            pl.BlockSpec((D, H), lambda i: (0, 0)),
            pl.BlockSpec((D, H), lambda i: (0, 0)),
        ],
        out_specs=[
            pl.BlockSpec((_NB, H), lambda i: (i, 0)),
            pl.BlockSpec((_NB, H), lambda i: (i, 0)),
        ],
        out_shape=[
            jax.ShapeDtypeStruct((N, H), jnp.float32),
            jax.ShapeDtypeStruct((N, H), jnp.float32),
        ],
    )(xm, Wmx, Wu)


def _mid_body(acc_ref, z_ref, ws_ref, bo_ref, wm_ref, wu_ref,
              z1_ref, zw_ref, zu_ref):
    a = acc_ref[0] + acc_ref[1]
    a += jnp.dot(z_ref[...], ws_ref[...], preferred_element_type=jnp.float32)
    z1 = jnp.maximum(a + bo_ref[0, :][None, :], 0.0)
    z1_ref[...] = z1
    zw_ref[...] = jnp.dot(z1, wm_ref[...], preferred_element_type=jnp.float32)
    zu_ref[...] = jnp.dot(z1, wu_ref[...], preferred_element_type=jnp.float32)


def _mid(acc, z, Ws, bo, Wmx, Wu):
    return pl.pallas_call(
        _mid_body,
        grid=(N // _NB,),
        in_specs=[
            pl.BlockSpec((NC, _NB, H), lambda i: (0, i, 0)),
            pl.BlockSpec((_NB, D), lambda i: (i, 0)),
            pl.BlockSpec((D, H), lambda i: (0, 0)),
            pl.BlockSpec((1, H), lambda i: (0, 0)),
            pl.BlockSpec((D, H), lambda i: (0, 0)),
            pl.BlockSpec((D, H), lambda i: (0, 0)),
        ],
        out_specs=[
            pl.BlockSpec((_NB, H), lambda i: (i, 0)),
            pl.BlockSpec((_NB, H), lambda i: (i, 0)),
            pl.BlockSpec((_NB, H), lambda i: (i, 0)),
        ],
        out_shape=[
            jax.ShapeDtypeStruct((N, H), jnp.float32),
            jax.ShapeDtypeStruct((N, H), jnp.float32),
            jax.ShapeDtypeStruct((N, H), jnp.float32),
        ],
    )(acc, z, Ws, bo.reshape(1, H), Wmx, Wu)


def _final_body(acc_ref, z_ref, ws_ref, bo_ref, o_ref):
    a = acc_ref[0] + acc_ref[1]
    a += jnp.dot(z_ref[...], ws_ref[...], preferred_element_type=jnp.float32)
    o_ref[...] = jnp.maximum(a + bo_ref[0, :][None, :], 0.0)


def _final(acc, z, Ws, bo):
    return pl.pallas_call(
        _final_body,
        grid=(N // _NB,),
        in_specs=[
            pl.BlockSpec((NC, _NB, H), lambda i: (0, i, 0)),
            pl.BlockSpec((_NB, D), lambda i: (i, 0)),
            pl.BlockSpec((D, H), lambda i: (0, 0)),
            pl.BlockSpec((1, H), lambda i: (0, 0)),
        ],
        out_specs=pl.BlockSpec((_NB, H), lambda i: (i, 0)),
        out_shape=jax.ShapeDtypeStruct((N, H), jnp.float32),
    )(acc, z, Ws, bo.reshape(1, H))


def kernel(x, edge_index, timestamps, time_diffs, unique_edges,
           timestamp_lists, te_w0, te_b0, te_w, te_b,
           Wm0, bm0, Wu0, Ws0, bo0, Wm1, bm1, Wu1, Ws1, bo1):
    src = edge_index[0]
    dst = edge_index[1]
    usrc = unique_edges[0]
    udst = unique_edges[1]

    c0 = _edge_const(timestamp_lists, time_diffs, te_w0, te_b0,
                     te_w, te_b, Wm0[D:], bm0)
    zw0, zu0 = _pre_nodes(x, Wm0[:D], Wu0)
    acc0 = _sc_layer()(zw0, zu0, c0, src, dst, usrc, udst)
    c1 = _edge_const(timestamp_lists, time_diffs, te_w0, te_b0,
                     te_w, te_b, Wm1[D:], bm1)
    z1, zw1, zu1 = _mid(acc0, x, Ws0, bo0, Wm1[:D], Wu1)
    acc1 = _sc_layer()(zw1, zu1, c1, src, dst, usrc, udst)
    return _final(acc1, z1, Ws1, bo1)
